# baseline (speedup 1.0000x reference)
import functools

import jax
import jax.numpy as jnp
import numpy as np
from jax.experimental import pallas as pl
from jax.experimental.pallas import tpu as pltpu


def _make_pool_kernel(scale):
    def _pool(x_ref, pw_ref, o_ref):
        x = x_ref[0]
        Cb, H, W = x.shape
        Ws = pw_ref.shape[1]
        Hs = H // scale
        y = jnp.dot(x.reshape(Cb * H, W), pw_ref[...],
                    preferred_element_type=jnp.float32)
        pooled = jnp.sum(y.reshape(Cb * Hs, scale, Ws), axis=1)
        o_ref[0] = pooled.reshape(Cb, Hs, Ws).astype(o_ref.dtype)

    return _pool


def _conv_kernel(p_ref, w_ref, o_ref):
    o_ref[0] = jnp.dot(w_ref[...], p_ref[0],
                       preferred_element_type=jnp.float32).astype(o_ref.dtype)


@functools.lru_cache(maxsize=32)
def _col_pool_operator(w_in, scale):
    ws = w_in // scale
    hit = (np.arange(w_in)[:, None] // scale) == np.arange(ws)[None, :]
    return hit.astype(np.float32) / float(scale * scale)


def _pick_cb(c, h, w_in, itemsize, target_bytes=16 * 2**20):
    img = h * w_in * itemsize
    best = 1
    for cb in range(1, c + 1):
        if c % cb == 0 and (cb * img <= target_bytes or best == 1):
            best = cb
    return best


def _run_pool_conv(x, w2d, *, scale):
    N, C, H, W = x.shape
    Hs, Ws = H // scale, W // scale
    C_out = w2d.shape[0]
    Cb = _pick_cb(C, H, W, x.dtype.itemsize)
    pw = jnp.asarray(_col_pool_operator(W, scale))

    pooled = pl.pallas_call(
        _make_pool_kernel(scale),
        out_shape=jax.ShapeDtypeStruct((N, C, Hs, Ws), x.dtype),
        grid=(N, C // Cb),
        in_specs=[
            pl.BlockSpec((1, Cb, H, W), lambda n, c: (n, c, 0, 0)),
            pl.BlockSpec((W, Ws), lambda n, c: (0, 0)),
        ],
        out_specs=pl.BlockSpec((1, Cb, Hs, Ws), lambda n, c: (n, c, 0, 0)),
        compiler_params=pltpu.CompilerParams(
            dimension_semantics=("parallel", "parallel"),
            vmem_limit_bytes=48 * 2**20,
        ),
        cost_estimate=pl.CostEstimate(
            flops=int(2 * N * C * H * W * Ws + N * C * Hs * Ws * scale),
            transcendentals=0,
            bytes_accessed=int(x.size * 4 + N * C * Hs * Ws * 4),
        ),
    )(x, pw)

    out = pl.pallas_call(
        _conv_kernel,
        out_shape=jax.ShapeDtypeStruct((N, C_out, Hs * Ws), x.dtype),
        grid=(N,),
        in_specs=[
            pl.BlockSpec((1, C, Hs * Ws), lambda n: (n, 0, 0)),
            pl.BlockSpec((C_out, C), lambda n: (0, 0)),
        ],
        out_specs=pl.BlockSpec((1, C_out, Hs * Ws), lambda n: (n, 0, 0)),
        compiler_params=pltpu.CompilerParams(
            dimension_semantics=("parallel",),
            vmem_limit_bytes=32 * 2**20,
        ),
        cost_estimate=pl.CostEstimate(
            flops=int(2 * N * C_out * C * Hs * Ws), transcendentals=0,
            bytes_accessed=int(N * (C + C_out) * Hs * Ws * 4),
        ),
    )(pooled.reshape(N, C, Hs * Ws), w2d)
    return out.reshape(N, C_out, Hs, Ws)


def kernel(hidden_states, weight, *, scale=8):
    five_d = hidden_states.ndim == 5
    if five_d:
        B, F, C, H, W = hidden_states.shape
        x = hidden_states.reshape(B * F, C, H, W)
    else:
        x = hidden_states
    C_out, C_in = weight.shape[0], weight.shape[1]
    w2d = weight.reshape(C_out, C_in).astype(x.dtype)
    out = _run_pool_conv(x, w2d, scale=scale)
    if five_d:
        out = out.reshape(B, F, C_out, out.shape[-2], out.shape[-1])
    return out


# speedup vs baseline: 1.0819x; 1.0819x over previous
import functools

import jax
import jax.numpy as jnp
import numpy as np
from jax.experimental import pallas as pl
from jax.experimental.pallas import tpu as pltpu


def _make_pool_kernel(scale):
    def _pool(x_ref, pw_ref, o_ref):
        x = x_ref[0]
        Cb, H, W = x.shape
        Ws = pw_ref.shape[1]
        Hs = H // scale
        y = jnp.dot(x.reshape(Cb * H, W), pw_ref[...],
                    preferred_element_type=jnp.float32)
        pooled = jnp.sum(y.reshape(Cb * Hs, scale, Ws), axis=1)
        o_ref[0] = pooled.reshape(Cb, Hs, Ws).astype(o_ref.dtype)

    return _pool


def _conv_kernel(p_ref, w_ref, o_ref):
    o_ref[0] = jnp.dot(w_ref[...], p_ref[0],
                       preferred_element_type=jnp.float32).astype(o_ref.dtype)


@functools.lru_cache(maxsize=32)
def _col_pool_operator(w_in, scale):
    ws = w_in // scale
    hit = (np.arange(w_in)[:, None] // scale) == np.arange(ws)[None, :]
    return hit.astype(np.float32) / float(scale * scale)


def _pick_cb(c, h, w_in, itemsize, target_bytes=16 * 2**20):
    img = h * w_in * itemsize
    best = 1
    for cb in range(1, c + 1):
        if c % cb == 0 and (cb * img <= target_bytes or best == 1):
            best = cb
    return best


def _run_pool_conv(x, w2d, *, scale):
    N, C, H, W = x.shape
    Hs, Ws = H // scale, W // scale
    C_out = w2d.shape[0]
    Cb = _pick_cb(C, H, W, x.dtype.itemsize)
    pw = jnp.asarray(_col_pool_operator(W, scale))

    pooled = pl.pallas_call(
        _make_pool_kernel(scale),
        out_shape=jax.ShapeDtypeStruct((N, C, Hs, Ws), jnp.bfloat16),
        grid=(N, C // Cb),
        in_specs=[
            pl.BlockSpec((1, Cb, H, W), lambda n, c: (n, c, 0, 0)),
            pl.BlockSpec((W, Ws), lambda n, c: (0, 0)),
        ],
        out_specs=pl.BlockSpec((1, Cb, Hs, Ws), lambda n, c: (n, c, 0, 0)),
        compiler_params=pltpu.CompilerParams(
            dimension_semantics=("parallel", "parallel"),
            vmem_limit_bytes=48 * 2**20,
        ),
        cost_estimate=pl.CostEstimate(
            flops=int(2 * N * C * H * W * Ws + N * C * Hs * Ws * scale),
            transcendentals=0,
            bytes_accessed=int(x.size * 4 + N * C * Hs * Ws * 4),
        ),
    )(x, pw)

    out = pl.pallas_call(
        _conv_kernel,
        out_shape=jax.ShapeDtypeStruct((N, C_out, Hs * Ws), x.dtype),
        grid=(N,),
        in_specs=[
            pl.BlockSpec((1, C, Hs * Ws), lambda n: (n, 0, 0)),
            pl.BlockSpec((C_out, C), lambda n: (0, 0)),
        ],
        out_specs=pl.BlockSpec((1, C_out, Hs * Ws), lambda n: (n, 0, 0)),
        compiler_params=pltpu.CompilerParams(
            dimension_semantics=("parallel",),
            vmem_limit_bytes=32 * 2**20,
        ),
        cost_estimate=pl.CostEstimate(
            flops=int(2 * N * C_out * C * Hs * Ws), transcendentals=0,
            bytes_accessed=int(N * (2 * C + 4 * C_out) * Hs * Ws),
        ),
    )(pooled.reshape(N, C, Hs * Ws), w2d)
    return out.reshape(N, C_out, Hs, Ws)


def kernel(hidden_states, weight, *, scale=8):
    five_d = hidden_states.ndim == 5
    if five_d:
        B, F, C, H, W = hidden_states.shape
        x = hidden_states.reshape(B * F, C, H, W)
    else:
        x = hidden_states
    C_out, C_in = weight.shape[0], weight.shape[1]
    w2d = weight.reshape(C_out, C_in).astype(jnp.bfloat16)
    out = _run_pool_conv(x, w2d, scale=scale)
    if five_d:
        out = out.reshape(B, F, C_out, out.shape[-2], out.shape[-1])
    return out


# speedup vs baseline: 1.0973x; 1.0142x over previous
import functools

import jax
import jax.numpy as jnp
import numpy as np
from jax.experimental import pallas as pl
from jax.experimental.pallas import tpu as pltpu


def _make_pool_kernel(scale):
    def _pool(x_ref, pw_ref, o_ref):
        x = x_ref[0]
        Cb, H, W = x.shape
        Ws = pw_ref.shape[1]
        Hs = H // scale
        y = jnp.dot(x.reshape(Cb * H, W), pw_ref[...],
                    preferred_element_type=jnp.float32)
        pooled = jnp.sum(y.reshape(Cb * Hs, scale, Ws), axis=1)
        o_ref[0] = pooled.reshape(Cb, Hs, Ws).astype(o_ref.dtype)

    return _pool


def _conv_kernel(p_ref, w_ref, o_ref):
    w = w_ref[...]
    for n in range(p_ref.shape[0]):
        o_ref[n] = jnp.dot(w, p_ref[n],
                           preferred_element_type=jnp.float32).astype(o_ref.dtype)


@functools.lru_cache(maxsize=32)
def _col_pool_operator(w_in, scale):
    ws = w_in // scale
    hit = (np.arange(w_in)[:, None] // scale) == np.arange(ws)[None, :]
    return hit.astype(np.float32) / float(scale * scale)


def _pick_cb(c, h, w_in, itemsize, target_bytes=16 * 2**20):
    img = h * w_in * itemsize
    best = 1
    for cb in range(1, c + 1):
        if c % cb == 0 and (cb * img <= target_bytes or best == 1):
            best = cb
    return best


def _run_pool_conv(x, w2d, *, scale):
    N, C, H, W = x.shape
    Hs, Ws = H // scale, W // scale
    C_out = w2d.shape[0]
    Cb = _pick_cb(C, H, W, x.dtype.itemsize)
    pw = jnp.asarray(_col_pool_operator(W, scale))

    pooled = pl.pallas_call(
        _make_pool_kernel(scale),
        out_shape=jax.ShapeDtypeStruct((N, C, Hs, Ws), jnp.bfloat16),
        grid=(N, C // Cb),
        in_specs=[
            pl.BlockSpec((1, Cb, H, W), lambda n, c: (n, c, 0, 0)),
            pl.BlockSpec((W, Ws), lambda n, c: (0, 0)),
        ],
        out_specs=pl.BlockSpec((1, Cb, Hs, Ws), lambda n, c: (n, c, 0, 0)),
        compiler_params=pltpu.CompilerParams(
            dimension_semantics=("parallel", "parallel"),
            vmem_limit_bytes=48 * 2**20,
        ),
        cost_estimate=pl.CostEstimate(
            flops=int(2 * N * C * H * W * Ws + N * C * Hs * Ws * scale),
            transcendentals=0,
            bytes_accessed=int(x.size * 4 + N * C * Hs * Ws * 4),
        ),
    )(x, pw)

    out = pl.pallas_call(
        _conv_kernel,
        out_shape=jax.ShapeDtypeStruct((N, C_out, Hs * Ws), x.dtype),
        grid=(1,),
        in_specs=[
            pl.BlockSpec((N, C, Hs * Ws), lambda i: (0, 0, 0)),
            pl.BlockSpec((C_out, C), lambda i: (0, 0)),
        ],
        out_specs=pl.BlockSpec((N, C_out, Hs * Ws), lambda i: (0, 0, 0)),
        compiler_params=pltpu.CompilerParams(
            dimension_semantics=("arbitrary",),
            vmem_limit_bytes=32 * 2**20,
        ),
        cost_estimate=pl.CostEstimate(
            flops=int(2 * N * C_out * C * Hs * Ws), transcendentals=0,
            bytes_accessed=int(N * (2 * C + 4 * C_out) * Hs * Ws),
        ),
    )(pooled.reshape(N, C, Hs * Ws), w2d)
    return out.reshape(N, C_out, Hs, Ws)


def kernel(hidden_states, weight, *, scale=8):
    five_d = hidden_states.ndim == 5
    if five_d:
        B, F, C, H, W = hidden_states.shape
        x = hidden_states.reshape(B * F, C, H, W)
    else:
        x = hidden_states
    C_out, C_in = weight.shape[0], weight.shape[1]
    w2d = weight.reshape(C_out, C_in).astype(jnp.bfloat16)
    out = _run_pool_conv(x, w2d, scale=scale)
    if five_d:
        out = out.reshape(B, F, C_out, out.shape[-2], out.shape[-1])
    return out
